# baseline (speedup 1.0000x reference)
import functools

import jax
import jax.numpy as jnp
from jax import lax
from jax.experimental import pallas as pl
from jax.experimental.pallas import tpu as pltpu
from jax.experimental.pallas import tpu_sc as plsc

N_DATA = 1000000
EMBED_DIM = 32
HIDDENS = 64
OUTPUT_DIM = 16
BATCH = 16384

_NC = 2
_NS = 16
_NW = _NC * _NS
_B_PER_W = BATCH // _NW


@functools.lru_cache(maxsize=None)
def _make_sc_gather():
  mesh = plsc.VectorSubcoreMesh(core_axis_name="c", subcore_axis_name="s")

  @functools.partial(
      pl.kernel,
      mesh=mesh,
      out_type=jax.ShapeDtypeStruct((BATCH, EMBED_DIM), jnp.float32),
      scratch_types=[
          pltpu.VMEM((_B_PER_W,), jnp.int32),
          pltpu.VMEM((_B_PER_W, EMBED_DIM), jnp.float32),
          pltpu.SemaphoreType.DMA,
      ],
      compiler_params=pltpu.CompilerParams(use_tc_tiling_on_sc=False),
  )
  def gather_kernel(table_hbm, idx_hbm, out_hbm, idx_v, rows_v, sem):
    wid = lax.axis_index("s") * _NC + lax.axis_index("c")
    base = wid * _B_PER_W
    pltpu.sync_copy(idx_hbm.at[pl.ds(base, _B_PER_W)], idx_v)
    pltpu.async_copy(table_hbm.at[idx_v], rows_v, sem).wait()
    pltpu.sync_copy(rows_v, out_hbm.at[pl.ds(base, _B_PER_W)])

  return gather_kernel


_BM = 2048


def _mlp_body(emb_ref, w1_ref, b1_ref, w2_ref, b2_ref, out_ref):
  emb = emb_ref[...]
  h = jnp.tanh(
      jnp.dot(emb, w1_ref[...], preferred_element_type=jnp.float32)
      + b1_ref[...]
  )
  z = (
      jnp.dot(h, w2_ref[...], preferred_element_type=jnp.float32)
      + b2_ref[...]
  )
  out_ref[...] = jax.nn.sigmoid(z)


def _mlp(emb, w1, b1, w2, b2):
  grid = (BATCH // _BM,)
  return pl.pallas_call(
      _mlp_body,
      grid=grid,
      in_specs=[
          pl.BlockSpec((_BM, EMBED_DIM), lambda i: (i, 0)),
          pl.BlockSpec((EMBED_DIM, HIDDENS), lambda i: (0, 0)),
          pl.BlockSpec((1, HIDDENS), lambda i: (0, 0)),
          pl.BlockSpec((HIDDENS, OUTPUT_DIM), lambda i: (0, 0)),
          pl.BlockSpec((1, OUTPUT_DIM), lambda i: (0, 0)),
      ],
      out_specs=pl.BlockSpec((_BM, OUTPUT_DIM), lambda i: (i, 0)),
      out_shape=jax.ShapeDtypeStruct((BATCH, OUTPUT_DIM), jnp.float32),
  )(emb, w1, b1, w2, b2)


@jax.jit
def kernel(idx, table, W1, b1, W2, b2):
  emb = _make_sc_gather()(table, idx.astype(jnp.int32))
  return _mlp(emb, W1, b1.reshape(1, HIDDENS), W2, b2.reshape(1, OUTPUT_DIM))
